# baseline (speedup 1.0000x reference)
import functools

import jax
import jax.numpy as jnp
from jax import lax
from jax.experimental import pallas as pl
from jax.experimental.pallas import tpu as pltpu
from jax.experimental.pallas import tpu_sc as plsc

K = 1024
D = 256
B = 8192
CHUNK = 1024
NCHUNK = B // CHUNK
TOL2 = 0.25

_NC = 2
_NS = 16
_NW = _NC * _NS
_BPW = K // _NW


def _col2row(v):
    n = v.shape[0]
    eye = (lax.broadcasted_iota(jnp.int32, (n, n), 0)
           == lax.broadcasted_iota(jnp.int32, (n, n), 1))
    filled = jnp.where(eye, jnp.broadcast_to(v, (n, n)), -jnp.inf)
    return jnp.max(filled, axis=0, keepdims=True)


def _row2col(v):
    n = v.shape[1]
    eye = (lax.broadcasted_iota(jnp.int32, (n, n), 0)
           == lax.broadcasted_iota(jnp.int32, (n, n), 1))
    filled = jnp.where(eye, jnp.broadcast_to(v, (n, n)), -jnp.inf)
    return jnp.max(filled, axis=1, keepdims=True)


def _gather_rows(table, idx):
    mesh = plsc.VectorSubcoreMesh(core_axis_name="c", subcore_axis_name="s")

    @functools.partial(
        pl.kernel,
        mesh=mesh,
        out_type=jax.ShapeDtypeStruct((K, D), jnp.float32),
        scratch_types=[
            pltpu.VMEM((_BPW,), jnp.int32),
            pltpu.VMEM((_BPW, D), jnp.float32),
            pltpu.SemaphoreType.DMA,
        ],
    )
    def gather_kernel(table_hbm, idx_hbm, out_hbm, idx_v, rows_v, sem):
        wid = lax.axis_index("s") * _NC + lax.axis_index("c")
        base = wid * _BPW
        pltpu.sync_copy(idx_hbm.at[pl.ds(base, _BPW)], idx_v)
        pltpu.async_copy(table_hbm.at[idx_v], rows_v, sem).wait()
        pltpu.sync_copy(rows_v, out_hbm.at[pl.ds(base, _BPW)])

    return gather_kernel(table, idx)


def _fused_body(x_ref, c_ref, repl_ref, pc_col_ref, pc_row_ref, batch_hbm,
                out_ref, counts_ref, sums_ref, c2r_ref, far_ref, xbuf, dsem):
    step = pl.program_id(0)

    @pl.when(step == 0)
    def _init():
        counts_ref[...] = jnp.zeros_like(counts_ref)
        sums_ref[...] = jnp.zeros_like(sums_ref)
        c = c_ref[...]
        c2r_ref[...] = _col2row(jnp.sum(c * c, axis=1, keepdims=True))

    @pl.when(step < NCHUNK)
    def _assign_step():
        x = x_ref[...]
        g = lax.dot_general(x, c_ref[...], (((1,), (1,)), ((), ())),
                            preferred_element_type=jnp.float32)
        d2 = c2r_ref[...] - 2.0 * g
        dmin = jnp.min(d2, axis=1, keepdims=True)
        col = lax.broadcasted_iota(jnp.int32, (CHUNK, K), 1)
        first = jnp.min(jnp.where(d2 == dmin, col, K), axis=1, keepdims=True)
        onehot = (col == first).astype(jnp.float32)
        counts_ref[...] += jnp.sum(onehot, axis=0, keepdims=True)
        sums_ref[...] += lax.dot_general(
            onehot.astype(jnp.bfloat16), x.astype(jnp.bfloat16),
            (((0,), (0,)), ((), ())),
            preferred_element_type=jnp.float32)

    @pl.when(step == NCHUNK)
    def _final_step():
        centers = c_ref[...]
        prev_c = pc_col_ref[...]
        prev_r = pc_row_ref[...]
        row_i = lax.broadcasted_iota(jnp.int32, (K, K), 0)
        col_i = lax.broadcasted_iota(jnp.int32, (K, K), 1)
        cb_r = counts_ref[...]
        cb_c = _row2col(cb_r)

        empty_c = (prev_c == 0.0) & (cb_c == 0.0)
        cb2_c = jnp.where(empty_c, 1.0, cb_c)
        empty_r = (prev_r == 0.0) & (cb_r == 0.0)
        cb2_r = jnp.where(empty_r, 1.0, cb_r)
        sums = jnp.where(empty_c, repl_ref[...], sums_ref[...])
        nc_c = prev_c + cb2_c
        nc_r = prev_r + cb2_r
        mask_c = cb2_c > 0.0
        safe_den = jnp.where(nc_c > 0.0, nc_c, 1.0)
        updated = (centers * prev_c + sums) / safe_den
        c1 = jnp.where(mask_c, updated, centers)
        out_ref[...] = c1

        g = lax.dot_general(c1, c1, (((1,), (1,)), ((), ())),
                            preferred_element_type=jnp.float32)
        c12_c = jnp.sum(c1 * c1, axis=1, keepdims=True)
        c12_r = _col2row(c12_c)
        p2 = c12_c + c12_r - 2.0 * g
        close = p2 < TOL2
        lose_hi = jnp.where(close & (col_i > row_i) & (nc_c <= nc_r),
                            1.0, 0.0)
        lose_lo = jnp.where(close & (col_i < row_i) & (nc_c < nc_r),
                            1.0, 0.0)
        lm_c = jnp.max(jnp.maximum(lose_hi, lose_lo),
                       axis=1, keepdims=True)
        nloser = jnp.sum(lm_c)

        @pl.when(nloser > 0.5)
        def _replace():
            def load_chunk(t):
                cp = pltpu.make_async_copy(
                    batch_hbm.at[pl.ds(t * CHUNK, CHUNK), :], xbuf, dsem)
                cp.start()
                cp.wait()
                return xbuf[...]

            def far_chunk(t, carry):
                x = load_chunk(t)
                a2_r = _col2row(jnp.sum(x * x, axis=1, keepdims=True))
                gt = lax.dot_general(c1, x, (((1,), (1,)), ((), ())),
                                     preferred_element_type=jnp.float32)
                d2 = a2_r + c12_c - 2.0 * gt
                far_ref[pl.ds(t, 1), :] = jnp.max(d2, axis=0, keepdims=True)
                return carry

            lax.fori_loop(0, NCHUNK, far_chunk, 0)
            far = far_ref[...]
            fif = (lax.broadcasted_iota(jnp.int32, (NCHUNK, CHUNK), 0) * CHUNK
                   + lax.broadcasted_iota(jnp.int32, (NCHUNK, CHUNK), 1)
                   ).astype(jnp.float32)

            def pick(r, carry):
                f, top = carry
                m = jnp.max(f)
                idx = jnp.min(jnp.where(f == m, fif, jnp.float32(B)))
                top = jnp.where(
                    lax.broadcasted_iota(jnp.int32, (K, 1), 0) == r, idx, top)
                f = jnp.where(fif == idx, -jnp.inf, f)
                return f, top

            _, top = lax.fori_loop(
                0, nloser.astype(jnp.int32), pick,
                (far, jnp.zeros((K, 1), jnp.float32)))

            ltri = (row_i >= col_i).astype(jnp.float32)
            ranks = lax.dot_general(ltri, lm_c, (((1,), (0,)), ((), ())),
                                    preferred_element_type=jnp.float32) - 1.0
            ranks = jnp.maximum(ranks, 0.0)
            oh = (ranks == col_i.astype(jnp.float32)).astype(jnp.float32)
            gidx = lax.dot_general(oh, top, (((1,), (0,)), ((), ())),
                                   preferred_element_type=jnp.float32)

            def repl_chunk(t, acc):
                x = load_chunk(t)
                base = lax.convert_element_type(t * CHUNK, jnp.float32)
                cols = lax.broadcasted_iota(
                    jnp.int32, (K, CHUNK), 1).astype(jnp.float32) + base
                sel = (gidx == cols).astype(jnp.float32)
                return acc + lax.dot_general(
                    sel, x, (((1,), (0,)), ((), ())),
                    preferred_element_type=jnp.float32)

            repl2 = lax.fori_loop(0, NCHUNK, repl_chunk,
                                  jnp.zeros((K, D), jnp.float32))
            out_ref[...] = jnp.where(lm_c > 0.5, repl2, out_ref[...])


def _fused(batch, centers, replacement, pc_col, pc_row):
    return pl.pallas_call(
        _fused_body,
        grid=(NCHUNK + 1,),
        in_specs=[
            pl.BlockSpec((CHUNK, D), lambda i: (jnp.minimum(i, NCHUNK - 1), 0)),
            pl.BlockSpec((K, D), lambda i: (0, 0)),
            pl.BlockSpec((K, D), lambda i: (0, 0)),
            pl.BlockSpec((K, 1), lambda i: (0, 0)),
            pl.BlockSpec((1, K), lambda i: (0, 0)),
            pl.BlockSpec(memory_space=pltpu.MemorySpace.HBM),
        ],
        out_specs=pl.BlockSpec((K, D), lambda i: (0, 0)),
        out_shape=jax.ShapeDtypeStruct((K, D), jnp.float32),
        scratch_shapes=[
            pltpu.VMEM((1, K), jnp.float32),
            pltpu.VMEM((K, D), jnp.float32),
            pltpu.VMEM((1, K), jnp.float32),
            pltpu.VMEM((NCHUNK, CHUNK), jnp.float32),
            pltpu.VMEM((CHUNK, D), jnp.float32),
            pltpu.SemaphoreType.DMA,
        ],
        compiler_params=pltpu.CompilerParams(
            dimension_semantics=("arbitrary",)),
    )(batch, centers, replacement, pc_col, pc_row, batch)


def kernel(batch, cluster_centers, cluster_counts):
    repl_idx = jax.random.randint(jax.random.key(1), (K,), 0, B)
    replacement = _gather_rows(batch, repl_idx.astype(jnp.int32))
    pc_col = cluster_counts.reshape(K, 1)
    pc_row = cluster_counts.reshape(1, K)
    return _fused(batch, cluster_centers, replacement, pc_col, pc_row)


# speedup vs baseline: 1.1289x; 1.1289x over previous
import functools

import jax
import jax.numpy as jnp
from jax import lax
from jax.experimental import pallas as pl
from jax.experimental.pallas import tpu as pltpu
from jax.experimental.pallas import tpu_sc as plsc

K = 1024
D = 256
B = 8192
CHUNK = 512
NCHUNK = B // CHUNK
TOL2 = 0.25

_NC = 2
_NS = 16
_NW = _NC * _NS
_BPW = K // _NW


def _col2row(v):
    n = v.shape[0]
    eye = (lax.broadcasted_iota(jnp.int32, (n, n), 0)
           == lax.broadcasted_iota(jnp.int32, (n, n), 1))
    filled = jnp.where(eye, jnp.broadcast_to(v, (n, n)), -jnp.inf)
    return jnp.max(filled, axis=0, keepdims=True)


def _row2col(v):
    n = v.shape[1]
    eye = (lax.broadcasted_iota(jnp.int32, (n, n), 0)
           == lax.broadcasted_iota(jnp.int32, (n, n), 1))
    filled = jnp.where(eye, jnp.broadcast_to(v, (n, n)), -jnp.inf)
    return jnp.max(filled, axis=1, keepdims=True)


def _gather_rows(table, idx):
    mesh = plsc.VectorSubcoreMesh(core_axis_name="c", subcore_axis_name="s")

    @functools.partial(
        pl.kernel,
        mesh=mesh,
        out_type=jax.ShapeDtypeStruct((K, D), jnp.float32),
        scratch_types=[
            pltpu.VMEM((_BPW,), jnp.int32),
            pltpu.VMEM((_BPW, D), jnp.float32),
            pltpu.SemaphoreType.DMA,
        ],
    )
    def gather_kernel(table_hbm, idx_hbm, out_hbm, idx_v, rows_v, sem):
        wid = lax.axis_index("s") * _NC + lax.axis_index("c")
        base = wid * _BPW
        pltpu.sync_copy(idx_hbm.at[pl.ds(base, _BPW)], idx_v)
        pltpu.async_copy(table_hbm.at[idx_v], rows_v, sem).wait()
        pltpu.sync_copy(rows_v, out_hbm.at[pl.ds(base, _BPW)])

    return gather_kernel(table, idx)


def _fused_body(x_ref, c_ref, repl_ref, pc_col_ref, pc_row_ref, batch_hbm,
                out_ref, counts_ref, sums_ref, c2r_ref, far_ref, xbuf, dsem):
    step = pl.program_id(0)

    @pl.when(step == 0)
    def _init():
        counts_ref[...] = jnp.zeros_like(counts_ref)
        sums_ref[...] = jnp.zeros_like(sums_ref)
        c = c_ref[...]
        c2r_ref[...] = _col2row(jnp.sum(c * c, axis=1, keepdims=True))

    @pl.when(step < NCHUNK)
    def _assign_step():
        x = x_ref[...]
        g = lax.dot_general(x, c_ref[...], (((1,), (1,)), ((), ())),
                            preferred_element_type=jnp.float32)
        d2 = c2r_ref[...] - 2.0 * g
        dmin = jnp.min(d2, axis=1, keepdims=True)
        onehot = jnp.where(d2 == dmin, 1.0, 0.0)
        counts_ref[...] += jnp.sum(onehot, axis=0, keepdims=True)
        sums_ref[...] += lax.dot_general(
            onehot.astype(jnp.bfloat16), x.astype(jnp.bfloat16),
            (((0,), (0,)), ((), ())),
            preferred_element_type=jnp.float32)

    @pl.when(step == NCHUNK)
    def _final_step():
        centers = c_ref[...]
        prev_c = pc_col_ref[...]
        prev_r = pc_row_ref[...]
        row_i = lax.broadcasted_iota(jnp.int32, (K, K), 0)
        col_i = lax.broadcasted_iota(jnp.int32, (K, K), 1)
        cb_r = counts_ref[...]
        cb_c = _row2col(cb_r)

        empty_c = (prev_c == 0.0) & (cb_c == 0.0)
        cb2_c = jnp.where(empty_c, 1.0, cb_c)
        empty_r = (prev_r == 0.0) & (cb_r == 0.0)
        cb2_r = jnp.where(empty_r, 1.0, cb_r)
        sums = jnp.where(empty_c, repl_ref[...], sums_ref[...])
        nc_c = prev_c + cb2_c
        nc_r = prev_r + cb2_r
        mask_c = cb2_c > 0.0
        safe_den = jnp.where(nc_c > 0.0, nc_c, 1.0)
        updated = (centers * prev_c + sums) / safe_den
        c1 = jnp.where(mask_c, updated, centers)
        out_ref[...] = c1

        g = lax.dot_general(c1, c1, (((1,), (1,)), ((), ())),
                            preferred_element_type=jnp.float32)
        c12_c = jnp.sum(c1 * c1, axis=1, keepdims=True)
        c12_r = _col2row(c12_c)
        p2 = c12_c + c12_r - 2.0 * g
        close = p2 < TOL2
        lose_hi = jnp.where(close & (col_i > row_i) & (nc_c <= nc_r),
                            1.0, 0.0)
        lose_lo = jnp.where(close & (col_i < row_i) & (nc_c < nc_r),
                            1.0, 0.0)
        lm_c = jnp.max(jnp.maximum(lose_hi, lose_lo),
                       axis=1, keepdims=True)
        nloser = jnp.sum(lm_c)

        @pl.when(nloser > 0.5)
        def _replace():
            def load_chunk(t):
                cp = pltpu.make_async_copy(
                    batch_hbm.at[pl.ds(t * CHUNK, CHUNK), :], xbuf, dsem)
                cp.start()
                cp.wait()
                return xbuf[...]

            def far_chunk(t, carry):
                x = load_chunk(t)
                a2_r = _col2row(jnp.sum(x * x, axis=1, keepdims=True))
                gt = lax.dot_general(c1, x, (((1,), (1,)), ((), ())),
                                     preferred_element_type=jnp.float32)
                d2 = a2_r + c12_c - 2.0 * gt
                far_ref[pl.ds(t, 1), :] = jnp.max(d2, axis=0, keepdims=True)
                return carry

            lax.fori_loop(0, NCHUNK, far_chunk, 0)
            far = far_ref[...]
            fif = (lax.broadcasted_iota(jnp.int32, (NCHUNK, CHUNK), 0) * CHUNK
                   + lax.broadcasted_iota(jnp.int32, (NCHUNK, CHUNK), 1)
                   ).astype(jnp.float32)

            def pick(r, carry):
                f, top = carry
                m = jnp.max(f)
                idx = jnp.min(jnp.where(f == m, fif, jnp.float32(B)))
                top = jnp.where(
                    lax.broadcasted_iota(jnp.int32, (K, 1), 0) == r, idx, top)
                f = jnp.where(fif == idx, -jnp.inf, f)
                return f, top

            _, top = lax.fori_loop(
                0, nloser.astype(jnp.int32), pick,
                (far, jnp.zeros((K, 1), jnp.float32)))

            ltri = (row_i >= col_i).astype(jnp.float32)
            ranks = lax.dot_general(ltri, lm_c, (((1,), (0,)), ((), ())),
                                    preferred_element_type=jnp.float32) - 1.0
            ranks = jnp.maximum(ranks, 0.0)
            oh = (ranks == col_i.astype(jnp.float32)).astype(jnp.float32)
            gidx = lax.dot_general(oh, top, (((1,), (0,)), ((), ())),
                                   preferred_element_type=jnp.float32)

            def repl_chunk(t, acc):
                x = load_chunk(t)
                base = lax.convert_element_type(t * CHUNK, jnp.float32)
                cols = lax.broadcasted_iota(
                    jnp.int32, (K, CHUNK), 1).astype(jnp.float32) + base
                sel = (gidx == cols).astype(jnp.float32)
                return acc + lax.dot_general(
                    sel, x, (((1,), (0,)), ((), ())),
                    preferred_element_type=jnp.float32)

            repl2 = lax.fori_loop(0, NCHUNK, repl_chunk,
                                  jnp.zeros((K, D), jnp.float32))
            out_ref[...] = jnp.where(lm_c > 0.5, repl2, out_ref[...])


def _fused(batch, centers, replacement, pc_col, pc_row):
    return pl.pallas_call(
        _fused_body,
        grid=(NCHUNK + 1,),
        in_specs=[
            pl.BlockSpec((CHUNK, D), lambda i: (jnp.minimum(i, NCHUNK - 1), 0)),
            pl.BlockSpec((K, D), lambda i: (0, 0)),
            pl.BlockSpec((K, D), lambda i: (0, 0)),
            pl.BlockSpec((K, 1), lambda i: (0, 0)),
            pl.BlockSpec((1, K), lambda i: (0, 0)),
            pl.BlockSpec(memory_space=pltpu.MemorySpace.HBM),
        ],
        out_specs=pl.BlockSpec((K, D), lambda i: (0, 0)),
        out_shape=jax.ShapeDtypeStruct((K, D), jnp.float32),
        scratch_shapes=[
            pltpu.VMEM((1, K), jnp.float32),
            pltpu.VMEM((K, D), jnp.float32),
            pltpu.VMEM((1, K), jnp.float32),
            pltpu.VMEM((NCHUNK, CHUNK), jnp.float32),
            pltpu.VMEM((CHUNK, D), jnp.float32),
            pltpu.SemaphoreType.DMA,
        ],
        compiler_params=pltpu.CompilerParams(
            dimension_semantics=("arbitrary",)),
    )(batch, centers, replacement, pc_col, pc_row, batch)


def kernel(batch, cluster_centers, cluster_counts):
    repl_idx = jax.random.randint(jax.random.key(1), (K,), 0, B)
    replacement = _gather_rows(batch, repl_idx.astype(jnp.int32))
    pc_col = cluster_counts.reshape(K, 1)
    pc_row = cluster_counts.reshape(1, K)
    return _fused(batch, cluster_centers, replacement, pc_col, pc_row)
